# Initial kernel scaffold; baseline (speedup 1.0000x reference)
import jax
import jax.numpy as jnp
from jax.experimental import pallas as pl


def kernel(temporal_embedding):
    raise NotImplementedError("write your pallas kernel here")


# speedup vs baseline: 6.5854x; 6.5854x over previous
import jax
import jax.numpy as jnp
from jax import lax
from jax.experimental import pallas as pl
from jax.experimental.pallas import tpu as pltpu
from jax.experimental.pallas import tpu_sc as plsc

F = 16
P = 196
N = F * P
LANES = 16
CHUNKS = N // LANES
K = 16
NUM_DMAS = 6
MK = 8


def _body(table_hbm, out_hbm, table_v, rows_v, mixed_v, sem, msem):
    c = lax.axis_index("c")
    s = lax.axis_index("s")
    start = 392 * (s // 2) + 96 * (2 * (s % 2) + c) + 8 * (s % 2)

    pltpu.sync_copy(table_hbm, table_v)

    lane = lax.iota(jnp.int32, LANES)
    zero = lane * 0
    hi = s + (F - 1)

    def build(i, _):
        col = lane + i * LANES
        idx = hi - col // P
        vals = plsc.load_gather(table_v, [idx, zero])
        for k in range(K):
            rows_v[k, pl.ds(i * LANES, LANES)] = vals
        return _

    lax.fori_loop(0, CHUNKS, build, 0)

    copies = [
        pltpu.async_copy(rows_v, out_hbm.at[pl.ds(start + j * K, K)], sem)
        for j in range(NUM_DMAS)
    ]

    @pl.when(s < 4)
    def _mixed():
        m = 2 * s + c
        hia = 2 * m + (F - 1)

        def mbuild(i, _):
            off = (lane + i * LANES) // P
            va = plsc.load_gather(table_v, [hia - off, zero])
            vb = plsc.load_gather(table_v, [hia + 1 - off, zero])
            for k in range(MK // 2):
                mixed_v[k, pl.ds(i * LANES, LANES)] = va
            for k in range(MK // 2, MK):
                mixed_v[k, pl.ds(i * LANES, LANES)] = vb
            return _

        lax.fori_loop(0, CHUNKS, mbuild, 0)
        pltpu.async_copy(
            mixed_v, out_hbm.at[pl.ds(392 * m + 192, MK)], msem
        ).wait()

    for cp in copies:
        cp.wait()


@jax.jit
def _fill(table):
    run = pl.kernel(
        _body,
        out_type=jax.ShapeDtypeStruct((N, N), jnp.float32),
        mesh=plsc.VectorSubcoreMesh(core_axis_name="c", subcore_axis_name="s"),
        compiler_params=pltpu.CompilerParams(needs_layout_passes=False),
        scratch_types=[
            pltpu.VMEM((2 * F - 1, 1), jnp.float32),
            pltpu.VMEM((K, N), jnp.float32),
            pltpu.VMEM((MK, N), jnp.float32),
            pltpu.SemaphoreType.DMA,
            pltpu.SemaphoreType.DMA,
        ],
    )
    return run(table)


def kernel(temporal_embedding):
    return _fill(temporal_embedding)
